# baseline (speedup 1.0000x reference)
import functools

import jax
import jax.numpy as jnp
from jax import lax
from jax.experimental import pallas as pl
from jax.experimental.pallas import tpu as pltpu
from jax.experimental.pallas import tpu_sc as plsc

_BS = 4096
_NS = 16
_NE = 8
_D = 64
_L = 16
_DC = _D // _L
_ROW_E = _NS * _NE * _D
_ROW_M = _NE * _NS
_ROW_N = _NS * _D
_ROW_A = _ROW_M + _ROW_N + _D
_ROW_O = _ROW_N + _D
_A_ENT = _ROW_M
_A_SELF = _ROW_M + _ROW_N


def _compute_row(ebuf, abuf, obuf):
    m_vecs = [abuf[pl.ds(e * _L, _L)] for e in range(_NE)]
    cnt = m_vecs[0]
    for e in range(1, _NE):
        cnt = cnt + m_vecs[e]
    inv = 1.0 / jnp.where(cnt == 0.0, 1.0, cnt)
    w_vecs = [m_vecs[e] * inv for e in range(_NE)]

    sv_acc = [None] * _DC
    for s in range(_NS):
        ws = [w_vecs[e][s] for e in range(_NE)]
        for dc in range(_DC):
            acc = ws[0] * ebuf[s, 0, pl.ds(dc * _L, _L)]
            for e in range(1, _NE):
                acc = acc + ws[e] * ebuf[s, e, pl.ds(dc * _L, _L)]
            nv = abuf[pl.ds(_A_ENT + s * _D + dc * _L, _L)] + acc
            obuf[pl.ds(s * _D + dc * _L, _L)] = nv
            sv_acc[dc] = nv if s == 0 else sv_acc[dc] + nv

    for dc in range(_DC):
        obuf[pl.ds(_ROW_N + dc * _L, _L)] = (
            abuf[pl.ds(_A_SELF + dc * _L, _L)] + sv_acc[dc] * (1.0 / _NS))


def _sc_body(edge_hbm, aux_hbm, out_hbm,
             ebuf0, ebuf1, abuf0, abuf1, obuf0, obuf1,
             si0, si1, so0, so1):
    info = plsc.get_sparse_core_info()
    nc = info.num_cores
    wid = lax.axis_index("s") * nc + lax.axis_index("c")
    nrows = _BS // (nc * info.num_subcores)
    base = wid * nrows
    npairs = nrows // 2

    def start_in(row, ebuf, abuf, si):
        pltpu.async_copy(edge_hbm.at[row, 0], ebuf, si)
        pltpu.async_copy(aux_hbm.at[row], abuf, si)

    def wait_in(row, ebuf, abuf, si):
        pltpu.make_async_copy(edge_hbm.at[row, 0], ebuf, si).wait()
        pltpu.make_async_copy(aux_hbm.at[row], abuf, si).wait()

    start_in(base, ebuf0, abuf0, si0)

    def pair_body(g, carry):
        r0 = base + 2 * g
        r1 = r0 + 1

        start_in(r1, ebuf1, abuf1, si1)
        wait_in(r0, ebuf0, abuf0, si0)

        @pl.when(g >= 1)
        def _():
            pltpu.make_async_copy(obuf0, out_hbm.at[r0 - 2], so0).wait()

        _compute_row(ebuf0, abuf0, obuf0)
        pltpu.async_copy(obuf0, out_hbm.at[r0], so0)

        @pl.when(g < npairs - 1)
        def _():
            start_in(r1 + 1, ebuf0, abuf0, si0)

        wait_in(r1, ebuf1, abuf1, si1)

        @pl.when(g >= 1)
        def _():
            pltpu.make_async_copy(obuf1, out_hbm.at[r1 - 2], so1).wait()

        _compute_row(ebuf1, abuf1, obuf1)
        pltpu.async_copy(obuf1, out_hbm.at[r1], so1)
        return carry

    lax.fori_loop(0, npairs, pair_body, 0)

    pltpu.make_async_copy(obuf0, out_hbm.at[base + nrows - 2], so0).wait()
    pltpu.make_async_copy(obuf1, out_hbm.at[base + nrows - 1], so1).wait()


@jax.jit
def _run(edge, aux):
    mesh = plsc.VectorSubcoreMesh(core_axis_name="c", subcore_axis_name="s")
    body = functools.partial(
        pl.kernel,
        mesh=mesh,
        out_type=jax.ShapeDtypeStruct((_BS, _ROW_O), jnp.float32),
        scratch_types=[
            pltpu.VMEM((_NS, _NE, _D), jnp.float32),
            pltpu.VMEM((_NS, _NE, _D), jnp.float32),
            pltpu.VMEM((_ROW_A,), jnp.float32),
            pltpu.VMEM((_ROW_A,), jnp.float32),
            pltpu.VMEM((_ROW_O,), jnp.float32),
            pltpu.VMEM((_ROW_O,), jnp.float32),
            pltpu.SemaphoreType.DMA,
            pltpu.SemaphoreType.DMA,
            pltpu.SemaphoreType.DMA,
            pltpu.SemaphoreType.DMA,
        ],
    )(_sc_body)
    return body(edge, aux)


def kernel(self_vectors, neighbor_entity_vectors, neighbor_edge_vectors, masks):
    bs = self_vectors.shape[0]
    maskt = jnp.swapaxes(masks.reshape(bs, _NS, _NE), 1, 2).reshape(bs, _ROW_M)
    aux = jnp.concatenate(
        [maskt,
         neighbor_entity_vectors.reshape(bs, _ROW_N),
         self_vectors.reshape(bs, _D)], axis=1)
    out = _run(neighbor_edge_vectors, aux)
    sv = out[:, _ROW_N:]
    nv = out[:, :_ROW_N]
    return (sv.reshape(bs, 1, _D), nv.reshape(bs, 1, _NS, _D))


# speedup vs baseline: 1.5484x; 1.5484x over previous
import functools

import jax
import jax.numpy as jnp
from jax import lax
from jax.experimental import pallas as pl
from jax.experimental.pallas import tpu as pltpu
from jax.experimental.pallas import tpu_sc as plsc

_BS = 4096
_S = 16
_E = 8
_D = 64
_L = 16
_DHI = 8
_DLO = 8
_TB = 32
_BL = 128
_G = _BL // _L
_NCHUNK = _S * _DHI


def _sc_kernel(edge6, mask4, ent5, self4,
               nv_out, sv_out,
               ebuf0, ebuf1, entb0, entb1, obuf0, obuf1,
               mbuf, invbuf, svacc, svbuf,
               si0, si1, so0, so1, sm):
    info = plsc.get_sparse_core_info()
    nc = info.num_cores
    tb = lax.axis_index("s") * nc + lax.axis_index("c")

    pltpu.async_copy(mask4.at[:, :, tb], mbuf, sm)
    pltpu.make_async_copy(mask4.at[:, :, tb], mbuf, sm).wait()

    def inv_body(s, c):
        for g in range(_G):
            cnt = mbuf[s, 0, pl.ds(g * _L, _L)]
            for e in range(1, _E):
                cnt = cnt + mbuf[s, e, pl.ds(g * _L, _L)]
            invbuf[s, pl.ds(g * _L, _L)] = 1.0 / jnp.where(cnt == 0.0, 1.0, cnt)
        return c
    lax.fori_loop(0, _S, inv_body, 0)

    def zbody(i, c):
        svacc[0, pl.ds(i * _L, _L)] = jnp.zeros((_L,), jnp.float32)
        return c
    lax.fori_loop(0, _DHI * _DLO * _BL // _L, zbody, 0)

    def start_in(s, dhi, ebuf, entb, si):
        pltpu.async_copy(edge6.at[s, :, dhi, tb], ebuf, si)
        pltpu.async_copy(ent5.at[s, dhi, tb], entb, si)

    def wait_in(s, dhi, ebuf, entb, si):
        pltpu.make_async_copy(edge6.at[s, :, dhi, tb], ebuf, si).wait()
        pltpu.make_async_copy(ent5.at[s, dhi, tb], entb, si).wait()

    def compute_chunk(s, dhi, ebuf, entb, obuf):
        for g in range(_G):
            inv = invbuf[s, pl.ds(g * _L, _L)]
            mv = [mbuf[s, e, pl.ds(g * _L, _L)] for e in range(_E)]
            for dlo in range(_DLO):
                acc = mv[0] * ebuf[0, dlo, pl.ds(g * _L, _L)]
                for e in range(1, _E):
                    acc = acc + mv[e] * ebuf[e, dlo, pl.ds(g * _L, _L)]
                nv = entb[dlo, pl.ds(g * _L, _L)] + acc * inv
                obuf[dlo, pl.ds(g * _L, _L)] = nv
                off = dhi * (_DLO * _BL) + dlo * _BL + g * _L
                plsc.addupdate(svacc.at[0, pl.ds(off, _L)], nv)

    start_in(0, 0, ebuf0, entb0, si0)
    start_in(0, 1, ebuf1, entb1, si1)

    def pair_body(j, carry):
        k0 = 2 * j
        k1 = k0 + 1
        s0 = k0 // _DHI
        d0 = lax.rem(k0, _DHI)
        s1 = k1 // _DHI
        d1 = lax.rem(k1, _DHI)

        wait_in(s0, d0, ebuf0, entb0, si0)

        @pl.when(j >= 1)
        def _():
            km = k0 - 2
            pltpu.make_async_copy(
                obuf0, nv_out.at[km // _DHI, lax.rem(km, _DHI), tb], so0).wait()

        compute_chunk(s0, d0, ebuf0, entb0, obuf0)
        pltpu.async_copy(obuf0, nv_out.at[s0, d0, tb], so0)

        @pl.when(k0 + 2 < _NCHUNK)
        def _():
            kn = k0 + 2
            start_in(kn // _DHI, lax.rem(kn, _DHI), ebuf0, entb0, si0)

        wait_in(s1, d1, ebuf1, entb1, si1)

        @pl.when(j >= 1)
        def _():
            km = k1 - 2
            pltpu.make_async_copy(
                obuf1, nv_out.at[km // _DHI, lax.rem(km, _DHI), tb], so1).wait()

        compute_chunk(s1, d1, ebuf1, entb1, obuf1)
        pltpu.async_copy(obuf1, nv_out.at[s1, d1, tb], so1)

        @pl.when(k1 + 2 < _NCHUNK)
        def _():
            kn = k1 + 2
            start_in(kn // _DHI, lax.rem(kn, _DHI), ebuf1, entb1, si1)

        return carry

    lax.fori_loop(0, _NCHUNK // 2, pair_body, 0)

    pltpu.make_async_copy(
        obuf0, nv_out.at[_S - 1, _DHI - 2, tb], so0).wait()
    pltpu.make_async_copy(
        obuf1, nv_out.at[_S - 1, _DHI - 1, tb], so1).wait()

    pltpu.async_copy(self4.at[:, tb], svbuf, sm)
    pltpu.make_async_copy(self4.at[:, tb], svbuf, sm).wait()

    def sv_body(i, c):
        dhi = i // _DLO
        dlo = lax.rem(i, _DLO)
        for g in range(_G):
            off = dhi * (_DLO * _BL) + dlo * _BL + g * _L
            v = (svbuf[dhi, dlo, pl.ds(g * _L, _L)]
                 + svacc[0, pl.ds(off, _L)] * (1.0 / _S))
            svbuf[dhi, dlo, pl.ds(g * _L, _L)] = v
        return c
    lax.fori_loop(0, _DHI * _DLO, sv_body, 0)

    pltpu.async_copy(svbuf, sv_out.at[:, tb], sm)
    pltpu.make_async_copy(svbuf, sv_out.at[:, tb], sm).wait()


@jax.jit
def _run(edge6, mask4, ent5, self4):
    mesh = plsc.VectorSubcoreMesh(core_axis_name="c", subcore_axis_name="s")
    body = functools.partial(
        pl.kernel,
        mesh=mesh,
        out_type=(
            jax.ShapeDtypeStruct((_S, _DHI, _TB, _DLO, _BL), jnp.float32),
            jax.ShapeDtypeStruct((_DHI, _TB, _DLO, _BL), jnp.float32),
        ),
        scratch_types=[
            pltpu.VMEM((_E, _DLO, _BL), jnp.float32),
            pltpu.VMEM((_E, _DLO, _BL), jnp.float32),
            pltpu.VMEM((_DLO, _BL), jnp.float32),
            pltpu.VMEM((_DLO, _BL), jnp.float32),
            pltpu.VMEM((_DLO, _BL), jnp.float32),
            pltpu.VMEM((_DLO, _BL), jnp.float32),
            pltpu.VMEM((_S, _E, _BL), jnp.float32),
            pltpu.VMEM((_S, _BL), jnp.float32),
            pltpu.VMEM((1, _DHI * _DLO * _BL), jnp.float32),
            pltpu.VMEM((_DHI, _DLO, _BL), jnp.float32),
            pltpu.SemaphoreType.DMA,
            pltpu.SemaphoreType.DMA,
            pltpu.SemaphoreType.DMA,
            pltpu.SemaphoreType.DMA,
            pltpu.SemaphoreType.DMA,
        ],
    )(_sc_kernel)
    return body(edge6, mask4, ent5, self4)


def kernel(self_vectors, neighbor_entity_vectors, neighbor_edge_vectors, masks):
    bs = self_vectors.shape[0]
    edge6 = (
        neighbor_edge_vectors.reshape(bs, _S, _E, _D)
        .transpose(1, 2, 3, 0)
        .reshape(_S, _E, _DHI, _DLO, _TB, _BL)
        .transpose(0, 1, 2, 4, 3, 5)
    )
    mask4 = (
        masks.reshape(bs, _S, _E)
        .transpose(1, 2, 0)
        .reshape(_S, _E, _TB, _BL)
    )
    ent5 = (
        neighbor_entity_vectors.reshape(bs, _S, _D)
        .transpose(1, 2, 0)
        .reshape(_S, _DHI, _DLO, _TB, _BL)
        .transpose(0, 1, 3, 2, 4)
    )
    self4 = (
        self_vectors.reshape(bs, _D)
        .transpose(1, 0)
        .reshape(_DHI, _DLO, _TB, _BL)
        .transpose(0, 2, 1, 3)
    )
    nv6, sv5 = _run(edge6, mask4, ent5, self4)
    nv = (
        nv6.transpose(0, 1, 3, 2, 4)
        .reshape(_S, _D, bs)
        .transpose(2, 0, 1)
        .reshape(bs, 1, _S, _D)
    )
    sv = (
        sv5.transpose(0, 2, 1, 3)
        .reshape(_D, bs)
        .transpose(1, 0)
        .reshape(bs, 1, _D)
    )
    return (sv, nv)


# speedup vs baseline: 1.7073x; 1.1026x over previous
import functools

import jax
import jax.numpy as jnp
from jax import lax
from jax.experimental import pallas as pl
from jax.experimental.pallas import tpu as pltpu
from jax.experimental.pallas import tpu_sc as plsc

_BS = 4096
_S = 16
_E = 8
_D = 64
_L = 16
_DHI = 8
_DLO = 8
_TB = 32
_BL = 128
_G = _BL // _L
_NCHUNK = _S * _DHI


def _sc_kernel(edge6, mask4, ent5, self4,
               nv_out, sv_out,
               ebuf0, ebuf1, entb0, entb1, obuf0, obuf1,
               mbuf, invbuf, svacc, svbuf,
               si0, si1, so0, so1, sm):
    info = plsc.get_sparse_core_info()
    nc = info.num_cores
    tb = lax.axis_index("s") * nc + lax.axis_index("c")

    pltpu.async_copy(mask4.at[:, :, tb], mbuf, sm)
    pltpu.make_async_copy(mask4.at[:, :, tb], mbuf, sm).wait()

    def inv_body(s, c):
        for g in range(_G):
            cnt = mbuf[s, 0, pl.ds(g * _L, _L)]
            for e in range(1, _E):
                cnt = cnt + mbuf[s, e, pl.ds(g * _L, _L)]
            invbuf[s, pl.ds(g * _L, _L)] = 1.0 / jnp.where(cnt == 0.0, 1.0, cnt)
        return c
    lax.fori_loop(0, _S, inv_body, 0)

    def zbody(i, c):
        svacc[0, pl.ds(i * _L, _L)] = jnp.zeros((_L,), jnp.float32)
        return c
    lax.fori_loop(0, _DHI * _DLO * _BL // _L, zbody, 0)

    def start_in(s, dhi, ebuf, entb, si):
        pltpu.async_copy(edge6.at[s, :, dhi, tb], ebuf, si)
        pltpu.async_copy(ent5.at[s, dhi, tb], entb, si)

    def wait_in(s, dhi, ebuf, entb, si):
        pltpu.make_async_copy(edge6.at[s, :, dhi, tb], ebuf, si).wait()
        pltpu.make_async_copy(ent5.at[s, dhi, tb], entb, si).wait()

    def compute_chunk(s, dhi, ebuf, entb, obuf):
        for g in range(_G):
            inv = invbuf[s, pl.ds(g * _L, _L)]
            wv = [mbuf[s, e, pl.ds(g * _L, _L)] * inv for e in range(_E)]
            for dlo in range(_DLO):
                p = [wv[e] * ebuf[e, dlo, pl.ds(g * _L, _L)]
                     for e in range(_E)]
                q = [p[0] + p[1], p[2] + p[3], p[4] + p[5], p[6] + p[7]]
                r = [q[0] + q[1], q[2] + q[3]]
                nv = (entb[dlo, pl.ds(g * _L, _L)] + r[0]) + r[1]
                obuf[dlo, pl.ds(g * _L, _L)] = nv
                off = dhi * (_DLO * _BL) + dlo * _BL + g * _L
                plsc.addupdate(svacc.at[0, pl.ds(off, _L)], nv)

    start_in(0, 0, ebuf0, entb0, si0)
    start_in(0, 1, ebuf1, entb1, si1)

    def pair_body(j, carry):
        k0 = 2 * j
        k1 = k0 + 1
        s0 = k0 // _DHI
        d0 = lax.rem(k0, _DHI)
        s1 = k1 // _DHI
        d1 = lax.rem(k1, _DHI)

        wait_in(s0, d0, ebuf0, entb0, si0)

        @pl.when(j >= 1)
        def _():
            km = k0 - 2
            pltpu.make_async_copy(
                obuf0, nv_out.at[km // _DHI, lax.rem(km, _DHI), tb], so0).wait()

        compute_chunk(s0, d0, ebuf0, entb0, obuf0)
        pltpu.async_copy(obuf0, nv_out.at[s0, d0, tb], so0)

        @pl.when(k0 + 2 < _NCHUNK)
        def _():
            kn = k0 + 2
            start_in(kn // _DHI, lax.rem(kn, _DHI), ebuf0, entb0, si0)

        wait_in(s1, d1, ebuf1, entb1, si1)

        @pl.when(j >= 1)
        def _():
            km = k1 - 2
            pltpu.make_async_copy(
                obuf1, nv_out.at[km // _DHI, lax.rem(km, _DHI), tb], so1).wait()

        compute_chunk(s1, d1, ebuf1, entb1, obuf1)
        pltpu.async_copy(obuf1, nv_out.at[s1, d1, tb], so1)

        @pl.when(k1 + 2 < _NCHUNK)
        def _():
            kn = k1 + 2
            start_in(kn // _DHI, lax.rem(kn, _DHI), ebuf1, entb1, si1)

        return carry

    lax.fori_loop(0, _NCHUNK // 2, pair_body, 0)

    pltpu.make_async_copy(
        obuf0, nv_out.at[_S - 1, _DHI - 2, tb], so0).wait()
    pltpu.make_async_copy(
        obuf1, nv_out.at[_S - 1, _DHI - 1, tb], so1).wait()

    pltpu.async_copy(self4.at[:, tb], svbuf, sm)
    pltpu.make_async_copy(self4.at[:, tb], svbuf, sm).wait()

    def sv_body(i, c):
        dhi = i // _DLO
        dlo = lax.rem(i, _DLO)
        for g in range(_G):
            off = dhi * (_DLO * _BL) + dlo * _BL + g * _L
            v = (svbuf[dhi, dlo, pl.ds(g * _L, _L)]
                 + svacc[0, pl.ds(off, _L)] * (1.0 / _S))
            svbuf[dhi, dlo, pl.ds(g * _L, _L)] = v
        return c
    lax.fori_loop(0, _DHI * _DLO, sv_body, 0)

    pltpu.async_copy(svbuf, sv_out.at[:, tb], sm)
    pltpu.make_async_copy(svbuf, sv_out.at[:, tb], sm).wait()


@jax.jit
def _run(edge6, mask4, ent5, self4):
    mesh = plsc.VectorSubcoreMesh(core_axis_name="c", subcore_axis_name="s")
    body = functools.partial(
        pl.kernel,
        mesh=mesh,
        out_type=(
            jax.ShapeDtypeStruct((_S, _DHI, _TB, _DLO, _BL), jnp.float32),
            jax.ShapeDtypeStruct((_DHI, _TB, _DLO, _BL), jnp.float32),
        ),
        scratch_types=[
            pltpu.VMEM((_E, _DLO, _BL), jnp.float32),
            pltpu.VMEM((_E, _DLO, _BL), jnp.float32),
            pltpu.VMEM((_DLO, _BL), jnp.float32),
            pltpu.VMEM((_DLO, _BL), jnp.float32),
            pltpu.VMEM((_DLO, _BL), jnp.float32),
            pltpu.VMEM((_DLO, _BL), jnp.float32),
            pltpu.VMEM((_S, _E, _BL), jnp.float32),
            pltpu.VMEM((_S, _BL), jnp.float32),
            pltpu.VMEM((1, _DHI * _DLO * _BL), jnp.float32),
            pltpu.VMEM((_DHI, _DLO, _BL), jnp.float32),
            pltpu.SemaphoreType.DMA,
            pltpu.SemaphoreType.DMA,
            pltpu.SemaphoreType.DMA,
            pltpu.SemaphoreType.DMA,
            pltpu.SemaphoreType.DMA,
        ],
    )(_sc_kernel)
    return body(edge6, mask4, ent5, self4)


def kernel(self_vectors, neighbor_entity_vectors, neighbor_edge_vectors, masks):
    bs = self_vectors.shape[0]
    edge6 = (
        neighbor_edge_vectors.reshape(bs, _S, _E, _D)
        .transpose(1, 2, 3, 0)
        .reshape(_S, _E, _DHI, _DLO, _TB, _BL)
        .transpose(0, 1, 2, 4, 3, 5)
    )
    mask4 = (
        masks.reshape(bs, _S, _E)
        .transpose(1, 2, 0)
        .reshape(_S, _E, _TB, _BL)
    )
    ent5 = (
        neighbor_entity_vectors.reshape(bs, _S, _D)
        .transpose(1, 2, 0)
        .reshape(_S, _DHI, _DLO, _TB, _BL)
        .transpose(0, 1, 3, 2, 4)
    )
    self4 = (
        self_vectors.reshape(bs, _D)
        .transpose(1, 0)
        .reshape(_DHI, _DLO, _TB, _BL)
        .transpose(0, 2, 1, 3)
    )
    nv6, sv5 = _run(edge6, mask4, ent5, self4)
    nv = (
        nv6.transpose(0, 1, 3, 2, 4)
        .reshape(_S, _D, bs)
        .transpose(2, 0, 1)
        .reshape(bs, 1, _S, _D)
    )
    sv = (
        sv5.transpose(0, 2, 1, 3)
        .reshape(_D, bs)
        .transpose(1, 0)
        .reshape(bs, 1, _D)
    )
    return (sv, nv)


# speedup vs baseline: 8.4007x; 4.9206x over previous
import functools

import jax
import jax.numpy as jnp
from jax import lax
from jax.experimental import pallas as pl
from jax.experimental.pallas import tpu as pltpu
from jax.experimental.pallas import tpu_sc as plsc

_BS = 4096
_S = 16
_E = 8
_D = 64
_L = 16
_DHI = 8
_DLO = 8
_TB = 32
_BL = 128
_G = _BL // _L
_NCHUNK = _S * _DHI


def _sc_kernel(edge6, mask4, ent5, self4,
               nv_out, sv_out,
               ebuf0, ebuf1, entb0, entb1, obuf0, obuf1,
               mbuf, invbuf, svacc, svbuf,
               si0, si1, so0, so1, sm):
    info = plsc.get_sparse_core_info()
    nc = info.num_cores
    tb = lax.axis_index("s") * nc + lax.axis_index("c")

    pltpu.async_copy(mask4.at[:, :, tb], mbuf, sm)
    pltpu.make_async_copy(mask4.at[:, :, tb], mbuf, sm).wait()

    def inv_body(s, c):
        for g in range(_G):
            cnt = mbuf[s, 0, pl.ds(g * _L, _L)]
            for e in range(1, _E):
                cnt = cnt + mbuf[s, e, pl.ds(g * _L, _L)]
            invbuf[s, pl.ds(g * _L, _L)] = 1.0 / jnp.where(cnt == 0.0, 1.0, cnt)
        return c
    lax.fori_loop(0, _S, inv_body, 0)

    def zbody(i, c):
        svacc[0, pl.ds(i * _L, _L)] = jnp.zeros((_L,), jnp.float32)
        return c
    lax.fori_loop(0, _DHI * _DLO * _BL // _L, zbody, 0)

    def start_in(s, dhi, ebuf, entb, si):
        pltpu.async_copy(edge6.at[s, :, dhi, tb], ebuf, si)
        pltpu.async_copy(ent5.at[s, dhi, tb], entb, si)

    def wait_in(s, dhi, ebuf, entb, si):
        pltpu.make_async_copy(edge6.at[s, :, dhi, tb], ebuf, si).wait()
        pltpu.make_async_copy(ent5.at[s, dhi, tb], entb, si).wait()

    def compute_chunk(s, dhi, ebuf, entb, obuf):
        for g in range(_G):
            inv = invbuf[s, pl.ds(g * _L, _L)]
            wv = [mbuf[s, e, pl.ds(g * _L, _L)] * inv for e in range(_E)]
            for dlo in range(_DLO):
                p = [wv[e] * ebuf[e, dlo, pl.ds(g * _L, _L)]
                     for e in range(_E)]
                q = [p[0] + p[1], p[2] + p[3], p[4] + p[5], p[6] + p[7]]
                r = [q[0] + q[1], q[2] + q[3]]
                nv = (entb[dlo, pl.ds(g * _L, _L)] + r[0]) + r[1]
                obuf[dlo, pl.ds(g * _L, _L)] = nv
                off = dhi * (_DLO * _BL) + dlo * _BL + g * _L
                plsc.addupdate(svacc.at[0, pl.ds(off, _L)], nv)

    start_in(0, 0, ebuf0, entb0, si0)
    start_in(0, 1, ebuf1, entb1, si1)

    def pair_body(j, carry):
        k0 = 2 * j
        k1 = k0 + 1
        s0 = k0 // _DHI
        d0 = lax.rem(k0, _DHI)
        s1 = k1 // _DHI
        d1 = lax.rem(k1, _DHI)

        wait_in(s0, d0, ebuf0, entb0, si0)

        @pl.when(j >= 1)
        def _():
            km = k0 - 2
            pltpu.make_async_copy(
                obuf0, nv_out.at[km // _DHI, lax.rem(km, _DHI), tb], so0).wait()

        compute_chunk(s0, d0, ebuf0, entb0, obuf0)
        pltpu.async_copy(obuf0, nv_out.at[s0, d0, tb], so0)

        @pl.when(k0 + 2 < _NCHUNK)
        def _():
            kn = k0 + 2
            start_in(kn // _DHI, lax.rem(kn, _DHI), ebuf0, entb0, si0)

        wait_in(s1, d1, ebuf1, entb1, si1)

        @pl.when(j >= 1)
        def _():
            km = k1 - 2
            pltpu.make_async_copy(
                obuf1, nv_out.at[km // _DHI, lax.rem(km, _DHI), tb], so1).wait()

        compute_chunk(s1, d1, ebuf1, entb1, obuf1)
        pltpu.async_copy(obuf1, nv_out.at[s1, d1, tb], so1)

        @pl.when(k1 + 2 < _NCHUNK)
        def _():
            kn = k1 + 2
            start_in(kn // _DHI, lax.rem(kn, _DHI), ebuf1, entb1, si1)

        return carry

    lax.fori_loop(0, _NCHUNK // 2, pair_body, 0)

    pltpu.make_async_copy(
        obuf0, nv_out.at[_S - 1, _DHI - 2, tb], so0).wait()
    pltpu.make_async_copy(
        obuf1, nv_out.at[_S - 1, _DHI - 1, tb], so1).wait()

    pltpu.async_copy(self4.at[:, tb], svbuf, sm)
    pltpu.make_async_copy(self4.at[:, tb], svbuf, sm).wait()

    def sv_body(i, c):
        dhi = i // _DLO
        dlo = lax.rem(i, _DLO)
        for g in range(_G):
            off = dhi * (_DLO * _BL) + dlo * _BL + g * _L
            v = (svbuf[dhi, dlo, pl.ds(g * _L, _L)]
                 + svacc[0, pl.ds(off, _L)] * (1.0 / _S))
            svbuf[dhi, dlo, pl.ds(g * _L, _L)] = v
        return c
    lax.fori_loop(0, _DHI * _DLO, sv_body, 0)

    pltpu.async_copy(svbuf, sv_out.at[:, tb], sm)
    pltpu.make_async_copy(svbuf, sv_out.at[:, tb], sm).wait()


def _tc_body(mask, edge, ent, self_, nv_out, sv_out):
    m = [mask[:, e, :] for e in range(_E)]
    cnt = m[0]
    for e in range(1, _E):
        cnt = cnt + m[e]
    inv = 1.0 / jnp.where(cnt == 0.0, 1.0, cnt)
    w = [m[e] * inv for e in range(_E)]
    agg = edge[:, 0] * w[0][:, None, :]
    for e in range(1, _E):
        agg = agg + edge[:, e] * w[e][:, None, :]
    nv = ent[...] + agg
    nv_out[...] = nv
    sacc = nv[0]
    for s in range(1, _S):
        sacc = sacc + nv[s]
    sv_out[...] = self_[...] + sacc * (1.0 / _S)


def _tc_call(mask_t, edge_t, ent_t, self_t, ntb):
    return pl.pallas_call(
        _tc_body,
        grid=(ntb,),
        in_specs=[
            pl.BlockSpec((_S, _E, _BL), lambda i: (0, 0, i)),
            pl.BlockSpec((_S, _E, _D, _BL), lambda i: (0, 0, 0, i)),
            pl.BlockSpec((_S, _D, _BL), lambda i: (0, 0, i)),
            pl.BlockSpec((_D, _BL), lambda i: (0, i)),
        ],
        out_specs=[
            pl.BlockSpec((_S, _D, _BL), lambda i: (0, 0, i)),
            pl.BlockSpec((_D, _BL), lambda i: (0, i)),
        ],
        out_shape=[
            jax.ShapeDtypeStruct((_S, _D, ntb * _BL), jnp.float32),
            jax.ShapeDtypeStruct((_D, ntb * _BL), jnp.float32),
        ],
    )(mask_t, edge_t, ent_t, self_t)


@jax.jit
def _run_tc(mask_t, edge_t, ent_t, self_t):
    return _tc_call(mask_t, edge_t, ent_t, self_t, _TB)


@jax.jit
def _run(edge6, mask4, ent5, self4):
    mesh = plsc.VectorSubcoreMesh(core_axis_name="c", subcore_axis_name="s")
    body = functools.partial(
        pl.kernel,
        mesh=mesh,
        out_type=(
            jax.ShapeDtypeStruct((_S, _DHI, _TB, _DLO, _BL), jnp.float32),
            jax.ShapeDtypeStruct((_DHI, _TB, _DLO, _BL), jnp.float32),
        ),
        scratch_types=[
            pltpu.VMEM((_E, _DLO, _BL), jnp.float32),
            pltpu.VMEM((_E, _DLO, _BL), jnp.float32),
            pltpu.VMEM((_DLO, _BL), jnp.float32),
            pltpu.VMEM((_DLO, _BL), jnp.float32),
            pltpu.VMEM((_DLO, _BL), jnp.float32),
            pltpu.VMEM((_DLO, _BL), jnp.float32),
            pltpu.VMEM((_S, _E, _BL), jnp.float32),
            pltpu.VMEM((_S, _BL), jnp.float32),
            pltpu.VMEM((1, _DHI * _DLO * _BL), jnp.float32),
            pltpu.VMEM((_DHI, _DLO, _BL), jnp.float32),
            pltpu.SemaphoreType.DMA,
            pltpu.SemaphoreType.DMA,
            pltpu.SemaphoreType.DMA,
            pltpu.SemaphoreType.DMA,
            pltpu.SemaphoreType.DMA,
        ],
    )(_sc_kernel)
    return body(edge6, mask4, ent5, self4)


def kernel(self_vectors, neighbor_entity_vectors, neighbor_edge_vectors, masks):
    bs = self_vectors.shape[0]
    mask_t = masks.reshape(bs, _S, _E).transpose(1, 2, 0)
    edge_t = neighbor_edge_vectors.reshape(bs, _S, _E, _D).transpose(1, 2, 3, 0)
    ent_t = neighbor_entity_vectors.reshape(bs, _S, _D).transpose(1, 2, 0)
    self_t = self_vectors.reshape(bs, _D).transpose(1, 0)
    nv_t, sv_t = _run_tc(mask_t, edge_t, ent_t, self_t)
    nv = nv_t.transpose(2, 0, 1).reshape(bs, 1, _S, _D)
    sv = sv_t.transpose(1, 0).reshape(bs, 1, _D)
    return (sv, nv)


def _kernel_sc_only(self_vectors, neighbor_entity_vectors, neighbor_edge_vectors, masks):
    bs = self_vectors.shape[0]
    edge6 = (
        neighbor_edge_vectors.reshape(bs, _S, _E, _D)
        .transpose(1, 2, 3, 0)
        .reshape(_S, _E, _DHI, _DLO, _TB, _BL)
        .transpose(0, 1, 2, 4, 3, 5)
    )
    mask4 = (
        masks.reshape(bs, _S, _E)
        .transpose(1, 2, 0)
        .reshape(_S, _E, _TB, _BL)
    )
    ent5 = (
        neighbor_entity_vectors.reshape(bs, _S, _D)
        .transpose(1, 2, 0)
        .reshape(_S, _DHI, _DLO, _TB, _BL)
        .transpose(0, 1, 3, 2, 4)
    )
    self4 = (
        self_vectors.reshape(bs, _D)
        .transpose(1, 0)
        .reshape(_DHI, _DLO, _TB, _BL)
        .transpose(0, 2, 1, 3)
    )
    nv6, sv5 = _run(edge6, mask4, ent5, self4)
    nv = (
        nv6.transpose(0, 1, 3, 2, 4)
        .reshape(_S, _D, bs)
        .transpose(2, 0, 1)
        .reshape(bs, 1, _S, _D)
    )
    sv = (
        sv5.transpose(0, 2, 1, 3)
        .reshape(_D, bs)
        .transpose(1, 0)
        .reshape(bs, 1, _D)
    )
    return (sv, nv)
